# Initial kernel scaffold; baseline (speedup 1.0000x reference)
import jax
import jax.numpy as jnp
from jax.experimental import pallas as pl


def kernel(x, input_pts, params):
    raise NotImplementedError("write your pallas kernel here")


# speedup vs baseline: 10.4031x; 10.4031x over previous
import functools

import jax
import jax.numpy as jnp
from jax import lax
from jax.experimental import pallas as pl
from jax.experimental.pallas import tpu as pltpu
from jax.experimental.pallas import tpu_sc as plsc

_F32_BIG = 3.0e38


def _knn_body(q_ref, rt_ref, o_ref, *, K, N, off):
    b = pl.program_id(0)
    q = q_ref[0]
    rt = rt_ref[0]
    d = None
    for dim in range(3):
        delta = q[:, dim:dim + 1] - rt[dim:dim + 1, :]
        sq = delta * delta
        d = sq if d is None else d + sq
    iota = lax.broadcasted_iota(jnp.int32, d.shape, 1)
    cols = []
    for _ in range(K):
        m = jnp.min(d, axis=1, keepdims=True)
        i = jnp.min(jnp.where(d == m, iota, jnp.int32(N)), axis=1,
                    keepdims=True)
        cols.append(i)
        d = jnp.where(iota == i, _F32_BIG, d)
    idx = jnp.concatenate(cols, axis=1)
    o_ref[0] = idx + b * off


def _knn(q, rt, K):
    B, M, _ = q.shape
    N = rt.shape[2]
    TQ = min(256, M)
    nmt = M // TQ
    body = functools.partial(_knn_body, K=K, N=N, off=N)
    return pl.pallas_call(
        body,
        grid=(B, nmt),
        in_specs=[
            pl.BlockSpec((1, TQ, 3), lambda b, m: (b, m, 0)),
            pl.BlockSpec((1, 3, N), lambda b, m: (b, 0, 0)),
        ],
        out_specs=pl.BlockSpec((1, TQ, K), lambda b, m: (b, m, 0)),
        out_shape=jax.ShapeDtypeStruct((B, M, K), jnp.int32),
    )(q, rt)


def _sc_gather(table, idx_flat):
    num, = idx_flat.shape
    W = table.shape[1]
    win = 128
    mesh = plsc.VectorSubcoreMesh(core_axis_name="core",
                                  subcore_axis_name="subcore")
    idx2 = idx_flat.reshape(1, num)

    @functools.partial(
        pl.kernel,
        out_type=jax.ShapeDtypeStruct((num, W), table.dtype),
        mesh=mesh)
    def gk(x_hbm, i_hbm, o_hbm):
        def body(i_vmem, o_vmem):
            pltpu.sync_copy(x_hbm.at[i_vmem.at[0]], o_vmem)
        pltpu.emit_pipeline(
            body,
            grid=(num // win,),
            in_specs=[pl.BlockSpec((1, win), index_map=lambda i: (0, i))],
            out_specs=[pl.BlockSpec((win, W), index_map=lambda i: (i, 0))],
            core_axis_name="subcore",
            dimension_semantics=(pltpu.PARALLEL,),
        )(i_hbm, o_hbm)

    return gk(table, idx2)


def _dense_math(g, q, cen, b1, w1, w2, b2, w3, b3, wr, *, K, C, Cout):
    bf = jnp.bfloat16
    TK = g.shape[0]
    T = TK // K
    g3 = g.reshape(T, K, g.shape[1])
    feat = g3[:, :, :C]
    pts = []
    for dim in range(3):
        pd = g3[:, :, C + dim:C + dim + 1] - q[:, dim:dim + 1][:, :, None]
        pts.append(pd)
    sq = pts[0] * pts[0] + pts[1] * pts[1] + pts[2] * pts[2]
    mx = jnp.sqrt(jnp.max(sq, axis=1))
    mx = jnp.where(mx == 0.0, 1.0, mx)
    dparts = []
    for dim in range(3):
        pn = pts[dim] / mx[:, :, None]
        dparts.append(pn - cen[dim:dim + 1, :][None])
    dists = jnp.concatenate(dparts, axis=2).reshape(TK, 48)
    h = jnp.maximum(
        jnp.dot(dists.astype(bf), w1, preferred_element_type=jnp.float32)
        + b1, 0.0)
    h = jnp.maximum(
        jnp.dot(h.astype(bf), w2, preferred_element_type=jnp.float32)
        + b2, 0.0)
    h = jnp.maximum(
        jnp.dot(h.astype(bf), w3, preferred_element_type=jnp.float32)
        + b3, 0.0)
    h3 = h.astype(bf).astype(jnp.float32).reshape(T, K, 16)
    featr = feat.astype(bf).astype(jnp.float32)
    acc = None
    for n in range(16):
        fw = jnp.sum(featr * h3[:, :, n:n + 1], axis=1)
        part = jnp.dot(fw.astype(bf), wr[n],
                       preferred_element_type=jnp.float32)
        acc = part if acc is None else acc + part
    return acc * (1.0 / K)


def _dense_big_body(g_ref, q_ref, cen_ref, w1_ref, b1_ref, w2_ref, b2_ref,
                    w3_ref, b3_ref, wr_ref, o_ref, *, K, C, Cout):
    o_ref[0] = _dense_math(
        g_ref[...], q_ref[0], cen_ref[...], b1_ref[...], w1_ref[...],
        w2_ref[...], b2_ref[...], w3_ref[...], b3_ref[...], wr_ref[...],
        K=K, C=C, Cout=Cout)


def _dense_big(g, q, prm, K, C, Cout):
    B, M, _ = q.shape
    T = min(256, M)
    nmt = M // T
    W = g.shape[1]
    cen, b1, w1, w2, b2, w3, b3, wr = prm
    body = functools.partial(_dense_big_body, K=K, C=C, Cout=Cout)
    return pl.pallas_call(
        body,
        grid=(B, nmt),
        in_specs=[
            pl.BlockSpec((T * K, W), lambda b, m: (b * nmt + m, 0)),
            pl.BlockSpec((1, T, 3), lambda b, m: (b, m, 0)),
            pl.BlockSpec((3, 16), lambda b, m: (0, 0)),
            pl.BlockSpec((48, 32), lambda b, m: (0, 0)),
            pl.BlockSpec((1, 32), lambda b, m: (0, 0)),
            pl.BlockSpec((32, 16), lambda b, m: (0, 0)),
            pl.BlockSpec((1, 16), lambda b, m: (0, 0)),
            pl.BlockSpec((16, 16), lambda b, m: (0, 0)),
            pl.BlockSpec((1, 16), lambda b, m: (0, 0)),
            pl.BlockSpec((16, C, Cout), lambda b, m: (0, 0, 0)),
        ],
        out_specs=pl.BlockSpec((1, T, Cout), lambda b, m: (b, m, 0)),
        out_shape=jax.ShapeDtypeStruct((B, M, Cout), jnp.float32),
    )(g, q, cen, w1, b1, w2, b2, w3, b3, wr)


def _dense_small_body(tab_ref, idx_ref, q_ref, cen_ref, w1_ref, b1_ref,
                      w2_ref, b2_ref, w3_ref, b3_ref, wr_ref, o_ref, *, K, C,
                      Cout, N):
    b = pl.program_id(0)
    tab = tab_ref[0]
    il = idx_ref[0] - b * N
    MK = il.shape[0]
    oh = (lax.broadcasted_iota(jnp.int32, (MK, N), 1) == il).astype(
        jnp.float32)
    g = jnp.dot(oh, tab, preferred_element_type=jnp.float32,
                precision=lax.Precision.HIGHEST)
    o_ref[0] = _dense_math(
        g, q_ref[0], cen_ref[...], b1_ref[...], w1_ref[...], w2_ref[...],
        b2_ref[...], w3_ref[...], b3_ref[...], wr_ref[...],
        K=K, C=C, Cout=Cout)


def _dense_small(tab, idx3, q, prm, K, C, Cout):
    B, N, W = tab.shape
    M = q.shape[1]
    MK = M * K
    cen, b1, w1, w2, b2, w3, b3, wr = prm
    body = functools.partial(_dense_small_body, K=K, C=C, Cout=Cout, N=N)
    return pl.pallas_call(
        body,
        grid=(B,),
        in_specs=[
            pl.BlockSpec((1, N, W), lambda b: (b, 0, 0)),
            pl.BlockSpec((1, MK, 1), lambda b: (b, 0, 0)),
            pl.BlockSpec((1, M, 3), lambda b: (b, 0, 0)),
            pl.BlockSpec((3, 16), lambda b: (0, 0)),
            pl.BlockSpec((48, 32), lambda b: (0, 0)),
            pl.BlockSpec((1, 32), lambda b: (0, 0)),
            pl.BlockSpec((32, 16), lambda b: (0, 0)),
            pl.BlockSpec((1, 16), lambda b: (0, 0)),
            pl.BlockSpec((16, 16), lambda b: (0, 0)),
            pl.BlockSpec((1, 16), lambda b: (0, 0)),
            pl.BlockSpec((16, C, Cout), lambda b: (0, 0, 0)),
        ],
        out_specs=pl.BlockSpec((1, M, Cout), lambda b: (b, 0, 0)),
        out_shape=jax.ShapeDtypeStruct((B, M, Cout), jnp.float32),
    )(tab, idx3, q, cen, w1, b1, w2, b2, w3, b3, wr)


def _bn_body(f_ref, g_ref, b_ref, o_ref):
    f = f_ref[...]
    m = jnp.mean(f, axis=0, keepdims=True)
    c = f - m
    v = jnp.mean(c * c, axis=0, keepdims=True)
    y = c / jnp.sqrt(v + 1e-5) * g_ref[...] + b_ref[...]
    o_ref[...] = jnp.maximum(y, 0.0)


def _bn(f, gamma, beta):
    B, M, C = f.shape
    out = pl.pallas_call(
        _bn_body,
        out_shape=jax.ShapeDtypeStruct((B * M, C), jnp.float32),
    )(f.reshape(B * M, C), gamma[None, :], beta[None, :])
    return out.reshape(B, M, C)


def _fc_body(x_ref, w_ref, b_ref, o_ref):
    o_ref[0] = (jnp.dot(x_ref[0].astype(jnp.bfloat16), w_ref[...],
                        preferred_element_type=jnp.float32) + b_ref[...])


def _fc(xin, wT, b):
    B, M, C = xin.shape
    O = wT.shape[1]
    return pl.pallas_call(
        _fc_body,
        grid=(B,),
        in_specs=[
            pl.BlockSpec((1, M, C), lambda bb: (bb, 0, 0)),
            pl.BlockSpec((C, O), lambda bb: (0, 0)),
            pl.BlockSpec((1, O), lambda bb: (0, 0)),
        ],
        out_specs=pl.BlockSpec((1, M, O), lambda bb: (bb, 0, 0)),
        out_shape=jax.ShapeDtypeStruct((B, M, O), jnp.float32),
    )(xin, wT, b)


def _prep_params(p):
    bf = jnp.bfloat16
    return (p["centers"],
            p["l1"]["b"][None, :],
            p["l1"]["w"].T.astype(bf),
            p["l2"]["w"].T.astype(bf), p["l2"]["b"][None, :],
            p["l3"]["w"].T.astype(bf), p["l3"]["b"][None, :],
            jnp.transpose(p["weight"], (1, 0, 2)).astype(bf))


def kernel(x, input_pts, params):
    P = input_pts
    B = x.shape[0]
    PT = jnp.swapaxes(P, 1, 2)

    def knn(M, N, K):
        return _knn(P[:, :M], PT[:, :, :N], K)

    idx0 = knn(4096, 4096, 16)
    idx1 = idx0[:, :2048]
    idx2 = knn(1024, 2048, 16)
    idx3 = knn(256, 1024, 16)
    idx4 = knn(64, 256, 8)
    idx5 = knn(16, 64, 8)
    idx6 = knn(8, 16, 4)
    idx5d = knn(16, 8, 4)
    idx4d = knn(64, 16, 4)
    idx3d = knn(256, 64, 4)
    idx2d = knn(1024, 256, 8)
    idx1d = knn(2048, 1024, 8)
    idx0d = knn(4096, 2048, 8)

    def make_table(xin, R, W):
        t = jnp.concatenate([xin, P[:, :R]], axis=-1)
        pad = W - t.shape[-1]
        if pad:
            t = jnp.pad(t, ((0, 0), (0, 0), (0, pad)))
        return t

    def block_sc(name, xin, R, M, K, idx):
        p = params[name]
        C = xin.shape[-1]
        Cout = p["weight"].shape[2]
        W = -(-(C + 3) // 128) * 128
        tab = make_table(xin, R, W).reshape(B * R, W)
        g = _sc_gather(tab, idx.reshape(-1))
        f = _dense_big(g, P[:, :M], _prep_params(p), K, C, Cout)
        bn = params["bn" + name[2:]]
        return _bn(f, bn["gamma"], bn["beta"])

    def block_tc(name, xin, R, M, K, idx):
        p = params[name]
        C = xin.shape[-1]
        Cout = p["weight"].shape[2]
        W = -(-(C + 3) // 16) * 16
        tab = make_table(xin, R, W)
        f = _dense_small(tab, idx.reshape(B, M * K, 1), P[:, :M],
                         _prep_params(p), K, C, Cout)
        bn = params["bn" + name[2:]]
        return _bn(f, bn["gamma"], bn["beta"])

    x0 = block_sc("cv0", x, 4096, 4096, 16, idx0)
    x1 = block_sc("cv1", x0, 4096, 2048, 16, idx1)
    x2 = block_sc("cv2", x1, 2048, 1024, 16, idx2)
    x3 = block_sc("cv3", x2, 1024, 256, 16, idx3)
    x4 = block_tc("cv4", x3, 256, 64, 8, idx4)
    x5 = block_tc("cv5", x4, 64, 16, 8, idx5)
    x6 = block_tc("cv6", x5, 16, 8, 4, idx6)
    x5d = block_tc("cv5d", x6, 8, 16, 4, idx5d)
    x5d = jnp.concatenate([x5d, x5], axis=2)
    x4d = block_tc("cv4d", x5d, 16, 64, 4, idx4d)
    x4d = jnp.concatenate([x4d, x4], axis=2)
    x3d = block_tc("cv3d", x4d, 64, 256, 4, idx3d)
    x3d = jnp.concatenate([x3d, x3], axis=2)
    x2d = block_sc("cv2d", x3d, 256, 1024, 8, idx2d)
    x2d = jnp.concatenate([x2d, x2], axis=2)
    x1d = block_sc("cv1d", x2d, 1024, 2048, 8, idx1d)
    x1d = jnp.concatenate([x1d, x1], axis=2)
    x0d = block_sc("cv0d", x1d, 2048, 4096, 8, idx0d)
    x0d = jnp.concatenate([x0d, x0], axis=2)
    return _fc(x0d, params["fcout"]["w"].T.astype(jnp.bfloat16),
               params["fcout"]["b"][None, :])
